# baseline (speedup 1.0000x reference)
import jax
import jax.numpy as jnp
from jax.experimental import pallas as pl
from jax.experimental.pallas import tpu as pltpu


def _pos_kernel(row_ref, col_ref, out_ref, scratch, sems):
    b, h, w, two_d = out_ref.shape
    d = two_d // 2
    ce = col_ref[:w, :]
    re = row_ref[:h, :]
    scratch[:, :, :d] = jnp.broadcast_to(ce[None, :, :], (h, w, d))
    scratch[:, :, d:] = jnp.broadcast_to(re[:, None, :], (h, w, d))
    for i in range(b):
        pltpu.make_async_copy(scratch, out_ref.at[i], sems.at[i]).start()
    for i in range(b):
        pltpu.make_async_copy(scratch, out_ref.at[i], sems.at[i]).wait()


def kernel(x, row_embed, col_embed):
    b = x.shape[0]
    h, w = x.shape[-2], x.shape[-1]
    d = row_embed.shape[-1]
    out = pl.pallas_call(
        _pos_kernel,
        in_specs=[
            pl.BlockSpec(row_embed.shape, lambda: (0, 0)),
            pl.BlockSpec(col_embed.shape, lambda: (0, 0)),
        ],
        out_specs=pl.BlockSpec(memory_space=pl.ANY),
        out_shape=jax.ShapeDtypeStruct((b, h, w, 2 * d), row_embed.dtype),
        scratch_shapes=[
            pltpu.VMEM((h, w, 2 * d), row_embed.dtype),
            pltpu.SemaphoreType.DMA((2 * b,)),
        ],
    )(row_embed, col_embed)
    return jnp.transpose(out, (0, 3, 1, 2))


# speedup vs baseline: 1.0576x; 1.0576x over previous
import jax
import jax.numpy as jnp
from jax.experimental import pallas as pl
from jax.experimental.pallas import tpu as pltpu


def _pos_kernel(row_ref, col_ref, out_ref, scratch, sems):
    b, h, w, two_d = out_ref.shape
    d = two_d // 2
    ce = col_ref[:w, :]
    re = row_ref[:h, :]
    d2 = 256
    scratch[:, :, :d] = jnp.broadcast_to(ce[None, :, :], (h, w, d))
    for i in range(b):
        pltpu.make_async_copy(
            scratch.at[:, :, :d], out_ref.at[i, :, :, :d], sems.at[i]).start()
    scratch[:, :, d:d + d2] = jnp.broadcast_to(re[:, None, :d2], (h, w, d2))
    for i in range(b):
        pltpu.make_async_copy(
            scratch.at[:, :, d:d + d2], out_ref.at[i, :, :, d:d + d2],
            sems.at[b + i]).start()
    scratch[:, :, d + d2:] = jnp.broadcast_to(re[:, None, d2:], (h, w, d - d2))
    for i in range(b):
        pltpu.make_async_copy(
            scratch.at[:, :, d + d2:], out_ref.at[i, :, :, d + d2:],
            sems.at[2 * b + i]).start()
    for i in range(b):
        pltpu.make_async_copy(
            scratch.at[:, :, :d], out_ref.at[i, :, :, :d], sems.at[i]).wait()
        pltpu.make_async_copy(
            scratch.at[:, :, d:d + d2], out_ref.at[i, :, :, d:d + d2],
            sems.at[b + i]).wait()
        pltpu.make_async_copy(
            scratch.at[:, :, d + d2:], out_ref.at[i, :, :, d + d2:],
            sems.at[2 * b + i]).wait()


def kernel(x, row_embed, col_embed):
    b = x.shape[0]
    h, w = x.shape[-2], x.shape[-1]
    d = row_embed.shape[-1]
    out = pl.pallas_call(
        _pos_kernel,
        in_specs=[
            pl.BlockSpec(row_embed.shape, lambda: (0, 0)),
            pl.BlockSpec(col_embed.shape, lambda: (0, 0)),
        ],
        out_specs=pl.BlockSpec(memory_space=pl.ANY),
        out_shape=jax.ShapeDtypeStruct((b, h, w, 2 * d), row_embed.dtype),
        scratch_shapes=[
            pltpu.VMEM((h, w, 2 * d), row_embed.dtype),
            pltpu.SemaphoreType.DMA((3 * b,)),
        ],
    )(row_embed, col_embed)
    return jnp.transpose(out, (0, 3, 1, 2))
